# baseline (speedup 1.0000x reference)
import functools

import jax
import jax.numpy as jnp
from jax import lax
from jax.experimental import pallas as pl
from jax.experimental.pallas import tpu as pltpu
from jax.experimental.pallas import tpu_sc as plsc

N_QUERY = 4
B = 65536
N_VERTS = 1024

_NC = 2
_NS = 16
_NW = _NC * _NS
_L = 16
_OUT_W = N_QUERY * 3

_B_SC = 39936
_B_TC = B - _B_SC
_TC_R = 1024


def _insert(carry, d, jv):
    m1, m2, m3, m4, i1, i2, i3, i4 = carry
    c1 = d > m1
    c2 = d > m2
    c3 = d > m3
    c4 = d > m4
    dm1 = jnp.minimum(m1, d)
    nm1 = jnp.maximum(m1, d)
    dm2 = jnp.minimum(m2, dm1)
    nm2 = jnp.maximum(m2, dm1)
    dm3 = jnp.minimum(m3, dm2)
    nm3 = jnp.maximum(m3, dm2)
    nm4 = jnp.maximum(m4, dm3)
    t1 = jnp.where(c1, i1, jv)
    ni1 = jnp.where(c1, jv, i1)
    t2 = jnp.where(c2, i2, t1)
    ni2 = jnp.where(c2, t1, i2)
    t3 = jnp.where(c3, i3, t2)
    ni3 = jnp.where(c3, t2, i3)
    ni4 = jnp.where(c4, t3, i4)
    return (nm1, nm2, nm3, nm4, ni1, ni2, ni3, ni4)


def _topk_chunk(chunk, carry, dxv, dyv, dzv, vx_v, vy_v, vz_v):
    jbase = chunk * _L
    vxc = vx_v[pl.ds(jbase, _L)]
    vyc = vy_v[pl.ds(jbase, _L)]
    vzc = vz_v[pl.ds(jbase, _L)]
    jb = jnp.zeros((_L,), jnp.int32) + jbase
    for t in range(_L):
        d = dxv * vxc[t] + dyv * vyc[t] + dzv * vzc[t]
        carry = _insert(carry, d, jb + t)
    return carry


def _sc_kernel(dx_h, dy_h, dz_h, vxb_h, vyb_h, vzb_h, vx_h, vy_h, vz_h, out_h,
               dx_v, dy_v, dz_v, vxb_v, vyb_v, vzb_v, vx_v, vy_v, vz_v, out_v,
               *, span):
    wid = lax.axis_index("s") * _NC + lax.axis_index("c")
    base = _B_TC + wid * span
    groups = span // _L
    pltpu.sync_copy(dx_h.at[pl.ds(base, span)], dx_v)
    pltpu.sync_copy(dy_h.at[pl.ds(base, span)], dy_v)
    pltpu.sync_copy(dz_h.at[pl.ds(base, span)], dz_v)
    pltpu.sync_copy(vxb_h, vxb_v)
    pltpu.sync_copy(vyb_h, vyb_v)
    pltpu.sync_copy(vzb_h, vzb_v)
    pltpu.sync_copy(vx_h, vx_v)
    pltpu.sync_copy(vy_h, vy_v)
    pltpu.sync_copy(vz_h, vz_v)

    iota = lax.iota(jnp.int32, _L)
    iota12 = iota * _OUT_W
    neg_inf = jnp.full((_L,), -jnp.inf, jnp.float32)
    zeros_i = jnp.zeros((_L,), jnp.int32)

    def group_body(g, _):
        dxv = dx_v[pl.ds(g * _L, _L)]
        dyv = dy_v[pl.ds(g * _L, _L)]
        dzv = dz_v[pl.ds(g * _L, _L)]
        init = (neg_inf, neg_inf, neg_inf, neg_inf,
                zeros_i, zeros_i, zeros_i, zeros_i)
        body = functools.partial(_topk_chunk, dxv=dxv, dyv=dyv, dzv=dzv,
                                 vx_v=vxb_v, vy_v=vyb_v, vz_v=vzb_v)
        res = lax.fori_loop(0, N_VERTS // _L, body, init)
        idxs = res[4:]
        gbase = g * (_L * _OUT_W)
        for s in range(N_QUERY):
            i_s = idxs[s]
            gx = plsc.load_gather(vx_v, [i_s])
            gy = plsc.load_gather(vy_v, [i_s])
            gz = plsc.load_gather(vz_v, [i_s])
            addr = iota12 + (gbase + s * 3)
            plsc.store_scatter(out_v, [addr], gx)
            plsc.store_scatter(out_v, [addr + 1], gy)
            plsc.store_scatter(out_v, [addr + 2], gz)
        return 0

    lax.fori_loop(0, groups, group_body, 0)
    pltpu.sync_copy(out_v, out_h.at[pl.ds(wid * span * _OUT_W, span * _OUT_W)])


def _run_sc(dx, dy, dz, vxb, vyb, vzb, vx, vy, vz):
    span = _B_SC // _NW
    mesh = plsc.VectorSubcoreMesh(core_axis_name="c", subcore_axis_name="s")
    run = pl.kernel(
        functools.partial(_sc_kernel, span=span),
        out_type=jax.ShapeDtypeStruct((_B_SC * _OUT_W,), jnp.float32),
        mesh=mesh,
        compiler_params=pltpu.CompilerParams(needs_layout_passes=False),
        scratch_types=[
            pltpu.VMEM((span,), jnp.float32),
            pltpu.VMEM((span,), jnp.float32),
            pltpu.VMEM((span,), jnp.float32),
            pltpu.VMEM((N_VERTS,), jnp.float32),
            pltpu.VMEM((N_VERTS,), jnp.float32),
            pltpu.VMEM((N_VERTS,), jnp.float32),
            pltpu.VMEM((N_VERTS,), jnp.float32),
            pltpu.VMEM((N_VERTS,), jnp.float32),
            pltpu.VMEM((N_VERTS,), jnp.float32),
            pltpu.VMEM((span * _OUT_W,), jnp.float32),
        ],
    )
    return run(dx, dy, dz, vxb, vyb, vzb, vx, vy, vz)


def _tc_kernel(d_ref, vt_ref, verts_ref, out_ref):
    d = d_ref[0]
    vt = vt_ref[...]
    verts = verts_ref[...]
    dots = lax.dot_general(d, vt, (((1,), (0,)), ((), ())),
                           preferred_element_type=jnp.float32)
    iota = lax.broadcasted_iota(jnp.int32, (_TC_R, N_VERTS), 1)
    sels = []
    for _ in range(N_QUERY):
        mx = jnp.max(dots, axis=1, keepdims=True)
        eq = dots == mx
        cand = jnp.where(eq, iota, N_VERTS)
        idx = jnp.min(cand, axis=1, keepdims=True)
        oh = iota == idx
        sel = lax.dot_general(oh.astype(jnp.float32), verts,
                              (((1,), (0,)), ((), ())),
                              precision=lax.Precision.HIGHEST,
                              preferred_element_type=jnp.float32)
        sels.append(sel.reshape(_TC_R, 1, 3))
        dots = jnp.where(oh, -jnp.inf, dots)
    out_ref[...] = jnp.concatenate(sels, axis=1)


def _run_tc(dirs3, vtb, verts):
    nblk = _B_TC // _TC_R
    return pl.pallas_call(
        _tc_kernel,
        grid=(nblk,),
        in_specs=[
            pl.BlockSpec((1, _TC_R, 3), lambda i: (i, 0, 0)),
            pl.BlockSpec((3, N_VERTS), lambda i: (0, 0)),
            pl.BlockSpec((N_VERTS, 3), lambda i: (0, 0)),
        ],
        out_specs=pl.BlockSpec((_TC_R, N_QUERY, 3), lambda i: (i, 0, 0)),
        out_shape=jax.ShapeDtypeStruct((_B_TC, N_QUERY, 3), jnp.float32),
    )(dirs3, vtb, verts)


def _round_bf16(x):
    u = lax.bitcast_convert_type(x, jnp.uint32)
    u = (u + jnp.uint32(0x7FFF) + ((u >> 16) & jnp.uint32(1))) & jnp.uint32(0xFFFF0000)
    return lax.bitcast_convert_type(u, jnp.float32)


def kernel(directions, vertices):
    dirs_b = _round_bf16(directions)
    verts_b = _round_bf16(vertices)

    parts = []
    if _B_SC:
        dx = dirs_b[:, 0]
        dy = dirs_b[:, 1]
        dz = dirs_b[:, 2]
        sc_out = _run_sc(dx, dy, dz,
                         verts_b[:, 0], verts_b[:, 1], verts_b[:, 2],
                         vertices[:, 0], vertices[:, 1], vertices[:, 2])
        parts.append(sc_out.reshape(_B_SC, N_QUERY, 3))
    if _B_TC:
        dirs3 = dirs_b[:_B_TC].reshape(_B_TC // _TC_R, _TC_R, 3)
        tc_out = _run_tc(dirs3, verts_b.T, vertices)
        parts.insert(0, tc_out)
    out = jnp.concatenate(parts, axis=0) if len(parts) > 1 else parts[0]
    return out


# speedup vs baseline: 1.0740x; 1.0740x over previous
import functools

import jax
import jax.numpy as jnp
from jax import lax
from jax.experimental import pallas as pl
from jax.experimental.pallas import tpu as pltpu
from jax.experimental.pallas import tpu_sc as plsc

N_QUERY = 4
B = 65536
N_VERTS = 1024

_NC = 2
_NS = 16
_NW = _NC * _NS
_L = 16
_OUT_W = N_QUERY * 3

_B_SC = 39936
_B_TC = B - _B_SC
_TC_R = 1024


def _insert(carry, d, jv):
    m1, m2, m3, m4, i1, i2, i3, i4 = carry
    c1 = d > m1
    c2 = d > m2
    c3 = d > m3
    c4 = d > m4
    dm1 = jnp.minimum(m1, d)
    nm1 = jnp.maximum(m1, d)
    dm2 = jnp.minimum(m2, dm1)
    nm2 = jnp.maximum(m2, dm1)
    dm3 = jnp.minimum(m3, dm2)
    nm3 = jnp.maximum(m3, dm2)
    nm4 = jnp.maximum(m4, dm3)
    t1 = jnp.where(c1, i1, jv)
    ni1 = jnp.where(c1, jv, i1)
    t2 = jnp.where(c2, i2, t1)
    ni2 = jnp.where(c2, t1, i2)
    t3 = jnp.where(c3, i3, t2)
    ni3 = jnp.where(c3, t2, i3)
    ni4 = jnp.where(c4, t3, i4)
    return (nm1, nm2, nm3, nm4, ni1, ni2, ni3, ni4)


def _topk_chunk(chunk, carry, dxv, dyv, dzv, vx_v, vy_v, vz_v):
    jbase = chunk * _L
    jb = jnp.zeros((_L,), jnp.int32) + jbase
    for t in range(_L):
        off = (jbase + t) * _L
        d = dxv * vx_v[pl.ds(off, _L)]
        d = d + dyv * vy_v[pl.ds(off, _L)]
        d = d + dzv * vz_v[pl.ds(off, _L)]
        carry = _insert(carry, d, jb + t)
    return carry


def _sc_kernel(dx_h, dy_h, dz_h, vxb_h, vyb_h, vzb_h, vx_h, vy_h, vz_h, out_h,
               dx_v, dy_v, dz_v, vxb_v, vyb_v, vzb_v, vx_v, vy_v, vz_v, out_v,
               *, span):
    wid = lax.axis_index("s") * _NC + lax.axis_index("c")
    base = _B_TC + wid * span
    groups = span // _L
    pltpu.sync_copy(dx_h.at[pl.ds(base, span)], dx_v)
    pltpu.sync_copy(dy_h.at[pl.ds(base, span)], dy_v)
    pltpu.sync_copy(dz_h.at[pl.ds(base, span)], dz_v)
    pltpu.sync_copy(vxb_h, vxb_v)
    pltpu.sync_copy(vyb_h, vyb_v)
    pltpu.sync_copy(vzb_h, vzb_v)
    pltpu.sync_copy(vx_h, vx_v)
    pltpu.sync_copy(vy_h, vy_v)
    pltpu.sync_copy(vz_h, vz_v)

    iota = lax.iota(jnp.int32, _L)
    iota12 = iota * _OUT_W
    neg_inf = jnp.full((_L,), -jnp.inf, jnp.float32)
    zeros_i = jnp.zeros((_L,), jnp.int32)

    def group_body(g, _):
        dxv = dx_v[pl.ds(g * _L, _L)]
        dyv = dy_v[pl.ds(g * _L, _L)]
        dzv = dz_v[pl.ds(g * _L, _L)]
        init = (neg_inf, neg_inf, neg_inf, neg_inf,
                zeros_i, zeros_i, zeros_i, zeros_i)
        body = functools.partial(_topk_chunk, dxv=dxv, dyv=dyv, dzv=dzv,
                                 vx_v=vxb_v, vy_v=vyb_v, vz_v=vzb_v)
        res = lax.fori_loop(0, N_VERTS // _L, body, init)
        idxs = res[4:]
        gbase = g * (_L * _OUT_W)
        for s in range(N_QUERY):
            i_s = idxs[s]
            gx = plsc.load_gather(vx_v, [i_s])
            gy = plsc.load_gather(vy_v, [i_s])
            gz = plsc.load_gather(vz_v, [i_s])
            addr = iota12 + (gbase + s * 3)
            plsc.store_scatter(out_v, [addr], gx)
            plsc.store_scatter(out_v, [addr + 1], gy)
            plsc.store_scatter(out_v, [addr + 2], gz)
        return 0

    lax.fori_loop(0, groups, group_body, 0)
    pltpu.sync_copy(out_v, out_h.at[pl.ds(wid * span * _OUT_W, span * _OUT_W)])


def _run_sc(dx, dy, dz, vxb, vyb, vzb, vx, vy, vz):
    span = _B_SC // _NW
    mesh = plsc.VectorSubcoreMesh(core_axis_name="c", subcore_axis_name="s")
    run = pl.kernel(
        functools.partial(_sc_kernel, span=span),
        out_type=jax.ShapeDtypeStruct((_B_SC * _OUT_W,), jnp.float32),
        mesh=mesh,
        compiler_params=pltpu.CompilerParams(needs_layout_passes=False),
        scratch_types=[
            pltpu.VMEM((span,), jnp.float32),
            pltpu.VMEM((span,), jnp.float32),
            pltpu.VMEM((span,), jnp.float32),
            pltpu.VMEM((N_VERTS * _L,), jnp.float32),
            pltpu.VMEM((N_VERTS * _L,), jnp.float32),
            pltpu.VMEM((N_VERTS * _L,), jnp.float32),
            pltpu.VMEM((N_VERTS,), jnp.float32),
            pltpu.VMEM((N_VERTS,), jnp.float32),
            pltpu.VMEM((N_VERTS,), jnp.float32),
            pltpu.VMEM((span * _OUT_W,), jnp.float32),
        ],
    )
    return run(dx, dy, dz, vxb, vyb, vzb, vx, vy, vz)


def _tc_kernel(d_ref, vt_ref, verts_ref, out_ref):
    d = d_ref[0]
    vt = vt_ref[...]
    verts = verts_ref[...]
    dots = lax.dot_general(d, vt, (((1,), (0,)), ((), ())),
                           preferred_element_type=jnp.float32)
    iota = lax.broadcasted_iota(jnp.int32, (_TC_R, N_VERTS), 1)
    sels = []
    for _ in range(N_QUERY):
        mx = jnp.max(dots, axis=1, keepdims=True)
        eq = dots == mx
        cand = jnp.where(eq, iota, N_VERTS)
        idx = jnp.min(cand, axis=1, keepdims=True)
        oh = iota == idx
        sel = lax.dot_general(oh.astype(jnp.float32), verts,
                              (((1,), (0,)), ((), ())),
                              precision=lax.Precision.HIGHEST,
                              preferred_element_type=jnp.float32)
        sels.append(sel)
        dots = jnp.where(oh, -jnp.inf, dots)
    out_ref[...] = jnp.concatenate(sels, axis=1)


def _run_tc(dirs3, vtb, verts):
    nblk = _B_TC // _TC_R
    return pl.pallas_call(
        _tc_kernel,
        grid=(nblk,),
        in_specs=[
            pl.BlockSpec((1, _TC_R, 3), lambda i: (i, 0, 0)),
            pl.BlockSpec((3, N_VERTS), lambda i: (0, 0)),
            pl.BlockSpec((N_VERTS, 3), lambda i: (0, 0)),
        ],
        out_specs=pl.BlockSpec((_TC_R, _OUT_W), lambda i: (i, 0)),
        out_shape=jax.ShapeDtypeStruct((_B_TC, _OUT_W), jnp.float32),
    )(dirs3, vtb, verts)


def _round_bf16(x):
    u = lax.bitcast_convert_type(x, jnp.uint32)
    u = (u + jnp.uint32(0x7FFF) + ((u >> 16) & jnp.uint32(1))) & jnp.uint32(0xFFFF0000)
    return lax.bitcast_convert_type(u, jnp.float32)


def kernel(directions, vertices):
    dirs_b = _round_bf16(directions)
    verts_b = _round_bf16(vertices)

    parts = []
    if _B_SC:
        dx = dirs_b[:, 0]
        dy = dirs_b[:, 1]
        dz = dirs_b[:, 2]
        vb_wide = jnp.broadcast_to(
            verts_b.T[:, :, None], (3, N_VERTS, _L)).reshape(3, N_VERTS * _L)
        sc_out = _run_sc(dx, dy, dz,
                         vb_wide[0], vb_wide[1], vb_wide[2],
                         vertices[:, 0], vertices[:, 1], vertices[:, 2])
        parts.append(sc_out.reshape(_B_SC, _OUT_W))
    if _B_TC:
        dirs3 = dirs_b[:_B_TC].reshape(_B_TC // _TC_R, _TC_R, 3)
        tc_out = _run_tc(dirs3, verts_b.T, vertices)
        parts.insert(0, tc_out)
    out = jnp.concatenate(parts, axis=0) if len(parts) > 1 else parts[0]
    return out.reshape(B, N_QUERY, 3)


# speedup vs baseline: 1.0978x; 1.0222x over previous
import functools

import jax
import jax.numpy as jnp
from jax import lax
from jax.experimental import pallas as pl
from jax.experimental.pallas import tpu as pltpu
from jax.experimental.pallas import tpu_sc as plsc

N_QUERY = 4
B = 65536
N_VERTS = 1024

_NC = 2
_NS = 16
_NW = _NC * _NS
_L = 16
_OUT_W = N_QUERY * 3

_B_SC = 40960
_B_TC = B - _B_SC
_TC_R = 1024


def _insert(carry, d, jv):
    m1, m2, m3, m4, i1, i2, i3, i4 = carry
    c1 = d > m1
    c2 = d > m2
    c3 = d > m3
    c4 = d > m4
    dm1 = jnp.minimum(m1, d)
    nm1 = jnp.maximum(m1, d)
    dm2 = jnp.minimum(m2, dm1)
    nm2 = jnp.maximum(m2, dm1)
    dm3 = jnp.minimum(m3, dm2)
    nm3 = jnp.maximum(m3, dm2)
    nm4 = jnp.maximum(m4, dm3)
    t1 = jnp.where(c1, i1, jv)
    ni1 = jnp.where(c1, jv, i1)
    t2 = jnp.where(c2, i2, t1)
    ni2 = jnp.where(c2, t1, i2)
    t3 = jnp.where(c3, i3, t2)
    ni3 = jnp.where(c3, t2, i3)
    ni4 = jnp.where(c4, t3, i4)
    return (nm1, nm2, nm3, nm4, ni1, ni2, ni3, ni4)


def _topk_chunk(chunk, carry, dxv, dyv, dzv, vx_v, vy_v, vz_v):
    jbase = chunk * _L
    jb = jnp.zeros((_L,), jnp.int32) + jbase
    for t in range(_L):
        off = (jbase + t) * _L
        d = dxv * vx_v[pl.ds(off, _L)]
        d = d + dyv * vy_v[pl.ds(off, _L)]
        d = d + dzv * vz_v[pl.ds(off, _L)]
        carry = _insert(carry, d, jb + t)
    return carry


def _sc_kernel(dx_h, dy_h, dz_h, vxb_h, vyb_h, vzb_h, vx_h, vy_h, vz_h, out_h,
               dx_v, dy_v, dz_v, vxb_v, vyb_v, vzb_v, vx_v, vy_v, vz_v, out_v,
               *, span):
    wid = lax.axis_index("s") * _NC + lax.axis_index("c")
    base = _B_TC + wid * span
    groups = span // _L
    pltpu.sync_copy(dx_h.at[pl.ds(base, span)], dx_v)
    pltpu.sync_copy(dy_h.at[pl.ds(base, span)], dy_v)
    pltpu.sync_copy(dz_h.at[pl.ds(base, span)], dz_v)
    pltpu.sync_copy(vxb_h, vxb_v)
    pltpu.sync_copy(vyb_h, vyb_v)
    pltpu.sync_copy(vzb_h, vzb_v)
    pltpu.sync_copy(vx_h, vx_v)
    pltpu.sync_copy(vy_h, vy_v)
    pltpu.sync_copy(vz_h, vz_v)

    iota = lax.iota(jnp.int32, _L)
    iota12 = iota * _OUT_W
    neg_inf = jnp.full((_L,), -jnp.inf, jnp.float32)
    zeros_i = jnp.zeros((_L,), jnp.int32)

    def group_body(g, _):
        dxv = dx_v[pl.ds(g * _L, _L)]
        dyv = dy_v[pl.ds(g * _L, _L)]
        dzv = dz_v[pl.ds(g * _L, _L)]
        init = (neg_inf, neg_inf, neg_inf, neg_inf,
                zeros_i, zeros_i, zeros_i, zeros_i)
        body = functools.partial(_topk_chunk, dxv=dxv, dyv=dyv, dzv=dzv,
                                 vx_v=vxb_v, vy_v=vyb_v, vz_v=vzb_v)
        res = lax.fori_loop(0, N_VERTS // _L, body, init)
        idxs = res[4:]
        gbase = g * (_L * _OUT_W)
        for s in range(N_QUERY):
            i_s = idxs[s]
            gx = plsc.load_gather(vx_v, [i_s])
            gy = plsc.load_gather(vy_v, [i_s])
            gz = plsc.load_gather(vz_v, [i_s])
            addr = iota12 + (gbase + s * 3)
            plsc.store_scatter(out_v, [addr], gx)
            plsc.store_scatter(out_v, [addr + 1], gy)
            plsc.store_scatter(out_v, [addr + 2], gz)
        return 0

    lax.fori_loop(0, groups, group_body, 0)
    pltpu.sync_copy(out_v, out_h.at[pl.ds(wid * span * _OUT_W, span * _OUT_W)])


def _run_sc(dx, dy, dz, vxb, vyb, vzb, vx, vy, vz):
    span = _B_SC // _NW
    mesh = plsc.VectorSubcoreMesh(core_axis_name="c", subcore_axis_name="s")
    run = pl.kernel(
        functools.partial(_sc_kernel, span=span),
        out_type=jax.ShapeDtypeStruct((_B_SC * _OUT_W,), jnp.float32),
        mesh=mesh,
        compiler_params=pltpu.CompilerParams(needs_layout_passes=False),
        scratch_types=[
            pltpu.VMEM((span,), jnp.float32),
            pltpu.VMEM((span,), jnp.float32),
            pltpu.VMEM((span,), jnp.float32),
            pltpu.VMEM((N_VERTS * _L,), jnp.float32),
            pltpu.VMEM((N_VERTS * _L,), jnp.float32),
            pltpu.VMEM((N_VERTS * _L,), jnp.float32),
            pltpu.VMEM((N_VERTS,), jnp.float32),
            pltpu.VMEM((N_VERTS,), jnp.float32),
            pltpu.VMEM((N_VERTS,), jnp.float32),
            pltpu.VMEM((span * _OUT_W,), jnp.float32),
        ],
    )
    return run(dx, dy, dz, vxb, vyb, vzb, vx, vy, vz)


def _tc_kernel(d_ref, vt_ref, verts_ref, out_ref):
    d = d_ref[0]
    vt = vt_ref[...]
    verts = verts_ref[...]
    dots = lax.dot_general(d, vt, (((1,), (0,)), ((), ())),
                           preferred_element_type=jnp.float32)
    iota = lax.broadcasted_iota(jnp.int32, (_TC_R, N_VERTS), 1)
    sels = []
    for _ in range(N_QUERY):
        mx = jnp.max(dots, axis=1, keepdims=True)
        eq = dots == mx
        cand = jnp.where(eq, iota, N_VERTS)
        idx = jnp.min(cand, axis=1, keepdims=True)
        oh = iota == idx
        sel = lax.dot_general(oh.astype(jnp.float32), verts,
                              (((1,), (0,)), ((), ())),
                              precision=lax.Precision.HIGHEST,
                              preferred_element_type=jnp.float32)
        sels.append(sel)
        dots = jnp.where(oh, -jnp.inf, dots)
    out_ref[...] = jnp.concatenate(sels, axis=1)


def _run_tc(dirs3, vtb, verts):
    nblk = _B_TC // _TC_R
    return pl.pallas_call(
        _tc_kernel,
        grid=(nblk,),
        in_specs=[
            pl.BlockSpec((1, _TC_R, 3), lambda i: (i, 0, 0)),
            pl.BlockSpec((3, N_VERTS), lambda i: (0, 0)),
            pl.BlockSpec((N_VERTS, 3), lambda i: (0, 0)),
        ],
        out_specs=pl.BlockSpec((_TC_R, _OUT_W), lambda i: (i, 0)),
        out_shape=jax.ShapeDtypeStruct((_B_TC, _OUT_W), jnp.float32),
    )(dirs3, vtb, verts)


def _round_bf16(x):
    u = lax.bitcast_convert_type(x, jnp.uint32)
    u = (u + jnp.uint32(0x7FFF) + ((u >> 16) & jnp.uint32(1))) & jnp.uint32(0xFFFF0000)
    return lax.bitcast_convert_type(u, jnp.float32)


def kernel(directions, vertices):
    dirs_b = _round_bf16(directions)
    verts_b = _round_bf16(vertices)

    parts = []
    if _B_SC:
        dx = dirs_b[:, 0]
        dy = dirs_b[:, 1]
        dz = dirs_b[:, 2]
        vb_wide = jnp.broadcast_to(
            verts_b.T[:, :, None], (3, N_VERTS, _L)).reshape(3, N_VERTS * _L)
        sc_out = _run_sc(dx, dy, dz,
                         vb_wide[0], vb_wide[1], vb_wide[2],
                         vertices[:, 0], vertices[:, 1], vertices[:, 2])
        parts.append(sc_out.reshape(_B_SC, _OUT_W))
    if _B_TC:
        dirs3 = dirs_b[:_B_TC].reshape(_B_TC // _TC_R, _TC_R, 3)
        tc_out = _run_tc(dirs3, verts_b.T, vertices)
        parts.insert(0, tc_out)
    out = jnp.concatenate(parts, axis=0) if len(parts) > 1 else parts[0]
    return out.reshape(B, N_QUERY, 3)


# speedup vs baseline: 1.1555x; 1.0526x over previous
import functools

import jax
import jax.numpy as jnp
from jax import lax
from jax.experimental import pallas as pl
from jax.experimental.pallas import tpu as pltpu
from jax.experimental.pallas import tpu_sc as plsc

N_QUERY = 4
B = 65536
N_VERTS = 1024

_NC = 2
_NS = 16
_NW = _NC * _NS
_L = 16
_OUT_W = N_QUERY * 3

_B_SC = 40960
_B_TC = B - _B_SC
_TC_R = 1024


def _insert(carry, d, jv):
    m1, m2, m3, m4, i1, i2, i3, i4 = carry
    c1 = d > m1
    c2 = d > m2
    c3 = d > m3
    c4 = d > m4
    dm1 = jnp.minimum(m1, d)
    nm1 = jnp.maximum(m1, d)
    dm2 = jnp.minimum(m2, dm1)
    nm2 = jnp.maximum(m2, dm1)
    dm3 = jnp.minimum(m3, dm2)
    nm3 = jnp.maximum(m3, dm2)
    nm4 = jnp.maximum(m4, dm3)
    t1 = jnp.where(c1, i1, jv)
    ni1 = jnp.where(c1, jv, i1)
    t2 = jnp.where(c2, i2, t1)
    ni2 = jnp.where(c2, t1, i2)
    t3 = jnp.where(c3, i3, t2)
    ni3 = jnp.where(c3, t2, i3)
    ni4 = jnp.where(c4, t3, i4)
    return (nm1, nm2, nm3, nm4, ni1, ni2, ni3, ni4)


def _topk_chunk(chunk, carry, dxv, dyv, dzv, vx_v, vy_v, vz_v):
    jbase = chunk * _L
    jb = jnp.zeros((_L,), jnp.int32) + jbase
    for t in range(_L):
        off = (jbase + t) * _L
        d = dxv * vx_v[pl.ds(off, _L)]
        d = d + dyv * vy_v[pl.ds(off, _L)]
        d = d + dzv * vz_v[pl.ds(off, _L)]
        carry = _insert(carry, d, jb + t)
    return carry


def _sc_kernel(dx_h, dy_h, dz_h, vxb_h, vyb_h, vzb_h, vx_h, vy_h, vz_h, out_h,
               dx_v, dy_v, dz_v, vxb_v, vyb_v, vzb_v, vx_v, vy_v, vz_v, out_v,
               *, span):
    wid = lax.axis_index("s") * _NC + lax.axis_index("c")
    base = _B_TC + wid * span
    groups = span // _L
    pltpu.sync_copy(dx_h.at[pl.ds(base, span)], dx_v)
    pltpu.sync_copy(dy_h.at[pl.ds(base, span)], dy_v)
    pltpu.sync_copy(dz_h.at[pl.ds(base, span)], dz_v)
    pltpu.sync_copy(vxb_h, vxb_v)
    pltpu.sync_copy(vyb_h, vyb_v)
    pltpu.sync_copy(vzb_h, vzb_v)
    pltpu.sync_copy(vx_h, vx_v)
    pltpu.sync_copy(vy_h, vy_v)
    pltpu.sync_copy(vz_h, vz_v)

    iota = lax.iota(jnp.int32, _L)
    iota12 = iota * _OUT_W
    neg_inf = jnp.full((_L,), -jnp.inf, jnp.float32)
    zeros_i = jnp.zeros((_L,), jnp.int32)

    def group_body(g, _):
        dxv = dx_v[pl.ds(g * _L, _L)]
        dyv = dy_v[pl.ds(g * _L, _L)]
        dzv = dz_v[pl.ds(g * _L, _L)]
        init = (neg_inf, neg_inf, neg_inf, neg_inf,
                zeros_i, zeros_i, zeros_i, zeros_i)
        body = functools.partial(_topk_chunk, dxv=dxv, dyv=dyv, dzv=dzv,
                                 vx_v=vxb_v, vy_v=vyb_v, vz_v=vzb_v)
        res = lax.fori_loop(0, N_VERTS // _L, body, init)
        idxs = res[4:]
        gbase = g * (_L * _OUT_W)
        for s in range(N_QUERY):
            i_s = idxs[s]
            gx = plsc.load_gather(vx_v, [i_s])
            gy = plsc.load_gather(vy_v, [i_s])
            gz = plsc.load_gather(vz_v, [i_s])
            addr = iota12 + (gbase + s * 3)
            plsc.store_scatter(out_v, [addr], gx)
            plsc.store_scatter(out_v, [addr + 1], gy)
            plsc.store_scatter(out_v, [addr + 2], gz)
        return 0

    lax.fori_loop(0, groups, group_body, 0)
    pltpu.sync_copy(out_v, out_h.at[pl.ds(wid * span * _OUT_W, span * _OUT_W)])


def _run_sc(dx, dy, dz, vxb, vyb, vzb, vx, vy, vz):
    span = _B_SC // _NW
    mesh = plsc.VectorSubcoreMesh(core_axis_name="c", subcore_axis_name="s")
    run = pl.kernel(
        functools.partial(_sc_kernel, span=span),
        out_type=jax.ShapeDtypeStruct((_B_SC * _OUT_W,), jnp.float32),
        mesh=mesh,
        compiler_params=pltpu.CompilerParams(needs_layout_passes=False),
        scratch_types=[
            pltpu.VMEM((span,), jnp.float32),
            pltpu.VMEM((span,), jnp.float32),
            pltpu.VMEM((span,), jnp.float32),
            pltpu.VMEM((N_VERTS * _L,), jnp.float32),
            pltpu.VMEM((N_VERTS * _L,), jnp.float32),
            pltpu.VMEM((N_VERTS * _L,), jnp.float32),
            pltpu.VMEM((N_VERTS,), jnp.float32),
            pltpu.VMEM((N_VERTS,), jnp.float32),
            pltpu.VMEM((N_VERTS,), jnp.float32),
            pltpu.VMEM((span * _OUT_W,), jnp.float32),
        ],
    )
    return run(dx, dy, dz, vxb, vyb, vzb, vx, vy, vz)


def _tc_kernel(d_ref, vt_ref, verts6_ref, out_ref):
    d = d_ref[0]
    vt = vt_ref[...]
    verts6 = verts6_ref[...]
    dots = lax.dot_general(d, vt, (((1,), (0,)), ((), ())),
                           preferred_element_type=jnp.float32)
    iota = lax.broadcasted_iota(jnp.int32, (_TC_R, N_VERTS), 1)
    sels = []
    for _ in range(N_QUERY):
        mx = jnp.max(dots, axis=1, keepdims=True)
        eq = dots == mx
        cand = jnp.where(eq, iota, N_VERTS)
        idx = jnp.min(cand, axis=1, keepdims=True)
        oh = iota == idx
        sel6 = lax.dot_general(oh.astype(jnp.float32), verts6,
                               (((1,), (0,)), ((), ())),
                               preferred_element_type=jnp.float32)
        sels.append(sel6[:, 0:3] + sel6[:, 3:6])
        dots = jnp.where(oh, -jnp.inf, dots)
    out_ref[...] = jnp.concatenate(sels, axis=1)


def _run_tc(dirs3, vtb, verts6):
    nblk = _B_TC // _TC_R
    return pl.pallas_call(
        _tc_kernel,
        grid=(nblk,),
        in_specs=[
            pl.BlockSpec((1, _TC_R, 3), lambda i: (i, 0, 0)),
            pl.BlockSpec((3, N_VERTS), lambda i: (0, 0)),
            pl.BlockSpec((N_VERTS, 6), lambda i: (0, 0)),
        ],
        out_specs=pl.BlockSpec((_TC_R, _OUT_W), lambda i: (i, 0)),
        out_shape=jax.ShapeDtypeStruct((_B_TC, _OUT_W), jnp.float32),
    )(dirs3, vtb, verts6)


def _round_bf16(x):
    u = lax.bitcast_convert_type(x, jnp.uint32)
    u = (u + jnp.uint32(0x7FFF) + ((u >> 16) & jnp.uint32(1))) & jnp.uint32(0xFFFF0000)
    return lax.bitcast_convert_type(u, jnp.float32)


def kernel(directions, vertices):
    dirs_b = _round_bf16(directions)
    verts_b = _round_bf16(vertices)

    parts = []
    if _B_SC:
        dx = dirs_b[:, 0]
        dy = dirs_b[:, 1]
        dz = dirs_b[:, 2]
        vb_wide = jnp.broadcast_to(
            verts_b.T[:, :, None], (3, N_VERTS, _L)).reshape(3, N_VERTS * _L)
        sc_out = _run_sc(dx, dy, dz,
                         vb_wide[0], vb_wide[1], vb_wide[2],
                         vertices[:, 0], vertices[:, 1], vertices[:, 2])
        parts.append(sc_out.reshape(_B_SC, _OUT_W))
    if _B_TC:
        dirs3 = dirs_b[:_B_TC].reshape(_B_TC // _TC_R, _TC_R, 3)
        verts6 = jnp.concatenate([verts_b, vertices - verts_b], axis=1)
        tc_out = _run_tc(dirs3, verts_b.T, verts6)
        parts.insert(0, tc_out)
    out = jnp.concatenate(parts, axis=0) if len(parts) > 1 else parts[0]
    return out.reshape(B, N_QUERY, 3)


# speedup vs baseline: 1.3761x; 1.1910x over previous
import functools

import jax
import jax.numpy as jnp
from jax import lax
from jax.experimental import pallas as pl
from jax.experimental.pallas import tpu as pltpu
from jax.experimental.pallas import tpu_sc as plsc

N_QUERY = 4
B = 65536
N_VERTS = 1024

_NC = 2
_NS = 16
_NW = _NC * _NS
_L = 16
_OUT_W = N_QUERY * 3

_B_SC = 40960
_B_TC = B - _B_SC
_TC_R = 1024


def _insert(carry, d, jv):
    m1, m2, m3, m4, i1, i2, i3, i4 = carry
    c1 = d > m1
    c2 = d > m2
    c3 = d > m3
    c4 = d > m4
    dm1 = jnp.minimum(m1, d)
    nm1 = jnp.maximum(m1, d)
    dm2 = jnp.minimum(m2, dm1)
    nm2 = jnp.maximum(m2, dm1)
    dm3 = jnp.minimum(m3, dm2)
    nm3 = jnp.maximum(m3, dm2)
    nm4 = jnp.maximum(m4, dm3)
    t1 = jnp.where(c1, i1, jv)
    ni1 = jnp.where(c1, jv, i1)
    t2 = jnp.where(c2, i2, t1)
    ni2 = jnp.where(c2, t1, i2)
    t3 = jnp.where(c3, i3, t2)
    ni3 = jnp.where(c3, t2, i3)
    ni4 = jnp.where(c4, t3, i4)
    return (nm1, nm2, nm3, nm4, ni1, ni2, ni3, ni4)


def _topk_chunk(chunk, carry, dxv, dyv, dzv, vx_v, vy_v, vz_v):
    jbase = chunk * _L
    jb = jnp.zeros((_L,), jnp.int32) + jbase
    for t in range(_L):
        off = (jbase + t) * _L
        d = dxv * vx_v[pl.ds(off, _L)]
        d = d + dyv * vy_v[pl.ds(off, _L)]
        d = d + dzv * vz_v[pl.ds(off, _L)]
        carry = _insert(carry, d, jb + t)
    return carry


def _sc_kernel(dx_h, dy_h, dz_h, vxb_h, vyb_h, vzb_h, vx_h, vy_h, vz_h, out_h,
               dx_v, dy_v, dz_v, vxb_v, vyb_v, vzb_v, vx_v, vy_v, vz_v, out_v,
               *, span):
    wid = lax.axis_index("s") * _NC + lax.axis_index("c")
    base = _B_TC + wid * span
    groups = span // _L
    pltpu.sync_copy(dx_h.at[pl.ds(base, span)], dx_v)
    pltpu.sync_copy(dy_h.at[pl.ds(base, span)], dy_v)
    pltpu.sync_copy(dz_h.at[pl.ds(base, span)], dz_v)
    pltpu.sync_copy(vxb_h, vxb_v)
    pltpu.sync_copy(vyb_h, vyb_v)
    pltpu.sync_copy(vzb_h, vzb_v)
    pltpu.sync_copy(vx_h, vx_v)
    pltpu.sync_copy(vy_h, vy_v)
    pltpu.sync_copy(vz_h, vz_v)

    iota = lax.iota(jnp.int32, _L)
    iota12 = iota * _OUT_W
    neg_inf = jnp.full((_L,), -jnp.inf, jnp.float32)
    zeros_i = jnp.zeros((_L,), jnp.int32)

    def group_body(g, _):
        dxv = dx_v[pl.ds(g * _L, _L)]
        dyv = dy_v[pl.ds(g * _L, _L)]
        dzv = dz_v[pl.ds(g * _L, _L)]
        init = (neg_inf, neg_inf, neg_inf, neg_inf,
                zeros_i, zeros_i, zeros_i, zeros_i)
        body = functools.partial(_topk_chunk, dxv=dxv, dyv=dyv, dzv=dzv,
                                 vx_v=vxb_v, vy_v=vyb_v, vz_v=vzb_v)
        res = lax.fori_loop(0, N_VERTS // _L, body, init)
        idxs = res[4:]
        gbase = g * (_L * _OUT_W)
        for s in range(N_QUERY):
            i_s = idxs[s]
            gx = plsc.load_gather(vx_v, [i_s])
            gy = plsc.load_gather(vy_v, [i_s])
            gz = plsc.load_gather(vz_v, [i_s])
            addr = iota12 + (gbase + s * 3)
            plsc.store_scatter(out_v, [addr], gx)
            plsc.store_scatter(out_v, [addr + 1], gy)
            plsc.store_scatter(out_v, [addr + 2], gz)
        return 0

    lax.fori_loop(0, groups, group_body, 0)
    pltpu.sync_copy(out_v, out_h.at[pl.ds(wid * span * _OUT_W, span * _OUT_W)])


def _run_sc(dx, dy, dz, vxb, vyb, vzb, vx, vy, vz):
    span = _B_SC // _NW
    mesh = plsc.VectorSubcoreMesh(core_axis_name="c", subcore_axis_name="s")
    run = pl.kernel(
        functools.partial(_sc_kernel, span=span),
        out_type=jax.ShapeDtypeStruct((_B_SC * _OUT_W,), jnp.float32),
        mesh=mesh,
        compiler_params=pltpu.CompilerParams(needs_layout_passes=False),
        scratch_types=[
            pltpu.VMEM((span,), jnp.float32),
            pltpu.VMEM((span,), jnp.float32),
            pltpu.VMEM((span,), jnp.float32),
            pltpu.VMEM((N_VERTS * _L,), jnp.float32),
            pltpu.VMEM((N_VERTS * _L,), jnp.float32),
            pltpu.VMEM((N_VERTS * _L,), jnp.float32),
            pltpu.VMEM((N_VERTS,), jnp.float32),
            pltpu.VMEM((N_VERTS,), jnp.float32),
            pltpu.VMEM((N_VERTS,), jnp.float32),
            pltpu.VMEM((span * _OUT_W,), jnp.float32),
        ],
    )
    return run(dx, dy, dz, vxb, vyb, vzb, vx, vy, vz)


def _tc_kernel(d_ref, vt_ref, verts6_ref, out_ref):
    d = d_ref[0]
    vt = vt_ref[...]
    verts6 = verts6_ref[...]
    dots = lax.dot_general(d, vt, (((1,), (0,)), ((), ())),
                           preferred_element_type=jnp.float32)
    iota = lax.broadcasted_iota(jnp.int32, (_TC_R, N_VERTS), 1)
    sels = []
    for _ in range(N_QUERY):
        mx = jnp.max(dots, axis=1, keepdims=True)
        eq = dots == mx
        cand = jnp.where(eq, iota, N_VERTS)
        idx = jnp.min(cand, axis=1, keepdims=True)
        oh = iota == idx
        sel6 = lax.dot_general(oh.astype(jnp.float32), verts6,
                               (((1,), (0,)), ((), ())),
                               preferred_element_type=jnp.float32)
        sels.append(sel6[:, 0:3] + sel6[:, 3:6])
        dots = jnp.where(oh, -jnp.inf, dots)
    out_ref[...] = jnp.concatenate(sels, axis=1)


def _run_tc(dirs3, vtb, verts6):
    nblk = _B_TC // _TC_R
    return pl.pallas_call(
        _tc_kernel,
        grid=(nblk,),
        in_specs=[
            pl.BlockSpec((1, _TC_R, 3), lambda i: (i, 0, 0)),
            pl.BlockSpec((3, N_VERTS), lambda i: (0, 0)),
            pl.BlockSpec((N_VERTS, 6), lambda i: (0, 0)),
        ],
        out_specs=pl.BlockSpec((_TC_R, _OUT_W), lambda i: (i, 0)),
        out_shape=jax.ShapeDtypeStruct((B, _OUT_W), jnp.float32),
    )(dirs3, vtb, verts6)


def _round_bf16(x):
    u = lax.bitcast_convert_type(x, jnp.uint32)
    u = (u + jnp.uint32(0x7FFF) + ((u >> 16) & jnp.uint32(1))) & jnp.uint32(0xFFFF0000)
    return lax.bitcast_convert_type(u, jnp.float32)


def kernel(directions, vertices):
    dirs_b = _round_bf16(directions)
    verts_b = _round_bf16(vertices)

    dx = dirs_b[:, 0]
    dy = dirs_b[:, 1]
    dz = dirs_b[:, 2]
    vb_wide = jnp.broadcast_to(
        verts_b.T[:, :, None], (3, N_VERTS, _L)).reshape(3, N_VERTS * _L)
    sc_out = _run_sc(dx, dy, dz,
                     vb_wide[0], vb_wide[1], vb_wide[2],
                     vertices[:, 0], vertices[:, 1], vertices[:, 2])
    dirs3 = dirs_b[:_B_TC].reshape(_B_TC // _TC_R, _TC_R, 3)
    verts6 = jnp.concatenate([verts_b, vertices - verts_b], axis=1)
    tc_out = _run_tc(dirs3, verts_b.T, verts6)
    out = lax.dynamic_update_slice(
        tc_out, sc_out.reshape(_B_SC, _OUT_W), (_B_TC, 0))
    return out.reshape(B, N_QUERY, 3)


# speedup vs baseline: 1.9507x; 1.4175x over previous
import functools

import jax
import jax.numpy as jnp
from jax import lax
from jax.experimental import pallas as pl
from jax.experimental.pallas import tpu as pltpu
from jax.experimental.pallas import tpu_sc as plsc

N_QUERY = 4
B = 65536
N_VERTS = 1024

_NC = 2
_NS = 16
_NW = _NC * _NS
_L = 16
_OUT_W = N_QUERY * 3

_B_SC = 26624
_B_TC = B - _B_SC
_TC_R = 1024


def _insert(carry, d, jv):
    m1, m2, m3, m4, i1, i2, i3, i4 = carry
    c1 = d > m1
    c2 = d > m2
    c3 = d > m3
    c4 = d > m4
    dm1 = jnp.minimum(m1, d)
    nm1 = jnp.maximum(m1, d)
    dm2 = jnp.minimum(m2, dm1)
    nm2 = jnp.maximum(m2, dm1)
    dm3 = jnp.minimum(m3, dm2)
    nm3 = jnp.maximum(m3, dm2)
    nm4 = jnp.maximum(m4, dm3)
    t1 = jnp.where(c1, i1, jv)
    ni1 = jnp.where(c1, jv, i1)
    t2 = jnp.where(c2, i2, t1)
    ni2 = jnp.where(c2, t1, i2)
    t3 = jnp.where(c3, i3, t2)
    ni3 = jnp.where(c3, t2, i3)
    ni4 = jnp.where(c4, t3, i4)
    return (nm1, nm2, nm3, nm4, ni1, ni2, ni3, ni4)


def _topk_chunk(chunk, carry, dxv, dyv, dzv, vx_v, vy_v, vz_v):
    jbase = chunk * _L
    jb = jnp.zeros((_L,), jnp.int32) + jbase
    for t in range(_L):
        off = (jbase + t) * _L
        d = dxv * vx_v[pl.ds(off, _L)]
        d = d + dyv * vy_v[pl.ds(off, _L)]
        d = d + dzv * vz_v[pl.ds(off, _L)]
        carry = _insert(carry, d, jb + t)
    return carry


def _sc_kernel(dx_h, dy_h, dz_h, vxb_h, vyb_h, vzb_h, vx_h, vy_h, vz_h, out_h,
               dx_v, dy_v, dz_v, vxb_v, vyb_v, vzb_v, vx_v, vy_v, vz_v, out_v,
               *, span):
    wid = lax.axis_index("s") * _NC + lax.axis_index("c")
    base = _B_TC + wid * span
    groups = span // _L
    pltpu.sync_copy(dx_h.at[pl.ds(base, span)], dx_v)
    pltpu.sync_copy(dy_h.at[pl.ds(base, span)], dy_v)
    pltpu.sync_copy(dz_h.at[pl.ds(base, span)], dz_v)
    pltpu.sync_copy(vxb_h, vxb_v)
    pltpu.sync_copy(vyb_h, vyb_v)
    pltpu.sync_copy(vzb_h, vzb_v)
    pltpu.sync_copy(vx_h, vx_v)
    pltpu.sync_copy(vy_h, vy_v)
    pltpu.sync_copy(vz_h, vz_v)

    iota = lax.iota(jnp.int32, _L)
    iota12 = iota * _OUT_W
    neg_inf = jnp.full((_L,), -jnp.inf, jnp.float32)
    zeros_i = jnp.zeros((_L,), jnp.int32)

    def group_body(g, _):
        dxv = dx_v[pl.ds(g * _L, _L)]
        dyv = dy_v[pl.ds(g * _L, _L)]
        dzv = dz_v[pl.ds(g * _L, _L)]
        init = (neg_inf, neg_inf, neg_inf, neg_inf,
                zeros_i, zeros_i, zeros_i, zeros_i)
        body = functools.partial(_topk_chunk, dxv=dxv, dyv=dyv, dzv=dzv,
                                 vx_v=vxb_v, vy_v=vyb_v, vz_v=vzb_v)
        res = lax.fori_loop(0, N_VERTS // _L, body, init)
        idxs = res[4:]
        gbase = g * (_L * _OUT_W)
        for s in range(N_QUERY):
            i_s = idxs[s]
            gx = plsc.load_gather(vx_v, [i_s])
            gy = plsc.load_gather(vy_v, [i_s])
            gz = plsc.load_gather(vz_v, [i_s])
            addr = iota12 + (gbase + s * 3)
            plsc.store_scatter(out_v, [addr], gx)
            plsc.store_scatter(out_v, [addr + 1], gy)
            plsc.store_scatter(out_v, [addr + 2], gz)
        return 0

    lax.fori_loop(0, groups, group_body, 0)
    pltpu.sync_copy(out_v, out_h.at[pl.ds(wid * span * _OUT_W, span * _OUT_W)])


def _run_sc(dx, dy, dz, vxb, vyb, vzb, vx, vy, vz):
    span = _B_SC // _NW
    mesh = plsc.VectorSubcoreMesh(core_axis_name="c", subcore_axis_name="s")
    run = pl.kernel(
        functools.partial(_sc_kernel, span=span),
        out_type=jax.ShapeDtypeStruct((_B_SC * _OUT_W,), jnp.float32),
        mesh=mesh,
        compiler_params=pltpu.CompilerParams(needs_layout_passes=False),
        scratch_types=[
            pltpu.VMEM((span,), jnp.float32),
            pltpu.VMEM((span,), jnp.float32),
            pltpu.VMEM((span,), jnp.float32),
            pltpu.VMEM((N_VERTS * _L,), jnp.float32),
            pltpu.VMEM((N_VERTS * _L,), jnp.float32),
            pltpu.VMEM((N_VERTS * _L,), jnp.float32),
            pltpu.VMEM((N_VERTS,), jnp.float32),
            pltpu.VMEM((N_VERTS,), jnp.float32),
            pltpu.VMEM((N_VERTS,), jnp.float32),
            pltpu.VMEM((span * _OUT_W,), jnp.float32),
        ],
    )
    return run(dx, dy, dz, vxb, vyb, vzb, vx, vy, vz)


def _tc_kernel(d_ref, vt_ref, verts6_ref, out_ref):
    d = d_ref[0]
    vt = vt_ref[...]
    verts6 = verts6_ref[...]
    dots = lax.dot_general(d, vt, (((1,), (0,)), ((), ())),
                           preferred_element_type=jnp.float32)
    iota = lax.broadcasted_iota(jnp.int32, (_TC_R, N_VERTS), 1)
    sels = []
    for _ in range(N_QUERY):
        mx = jnp.max(dots, axis=1, keepdims=True)
        eq = dots == mx
        cand = jnp.where(eq, iota, N_VERTS)
        idx = jnp.min(cand, axis=1, keepdims=True)
        oh = iota == idx
        sel6 = lax.dot_general(oh.astype(jnp.float32), verts6,
                               (((1,), (0,)), ((), ())),
                               preferred_element_type=jnp.float32)
        sels.append(sel6[:, 0:3] + sel6[:, 3:6])
        dots = jnp.where(oh, -jnp.inf, dots)
    out_ref[...] = jnp.concatenate(sels, axis=1)


def _run_tc(dirs3, vtb, verts6):
    nblk = _B_TC // _TC_R
    return pl.pallas_call(
        _tc_kernel,
        grid=(nblk,),
        in_specs=[
            pl.BlockSpec((1, _TC_R, 3), lambda i: (i, 0, 0)),
            pl.BlockSpec((3, N_VERTS), lambda i: (0, 0)),
            pl.BlockSpec((N_VERTS, 6), lambda i: (0, 0)),
        ],
        out_specs=pl.BlockSpec((_TC_R, _OUT_W), lambda i: (i, 0)),
        out_shape=jax.ShapeDtypeStruct((B, _OUT_W), jnp.float32),
    )(dirs3, vtb, verts6)


def _round_bf16(x):
    u = lax.bitcast_convert_type(x, jnp.uint32)
    u = (u + jnp.uint32(0x7FFF) + ((u >> 16) & jnp.uint32(1))) & jnp.uint32(0xFFFF0000)
    return lax.bitcast_convert_type(u, jnp.float32)


def kernel(directions, vertices):
    dirs_b = _round_bf16(directions)
    verts_b = _round_bf16(vertices)

    dx = dirs_b[:, 0]
    dy = dirs_b[:, 1]
    dz = dirs_b[:, 2]
    vb_wide = jnp.broadcast_to(
        verts_b.T[:, :, None], (3, N_VERTS, _L)).reshape(3, N_VERTS * _L)
    sc_out = _run_sc(dx, dy, dz,
                     vb_wide[0], vb_wide[1], vb_wide[2],
                     vertices[:, 0], vertices[:, 1], vertices[:, 2])
    dirs3 = dirs_b[:_B_TC].reshape(_B_TC // _TC_R, _TC_R, 3)
    verts6 = jnp.concatenate([verts_b, vertices - verts_b], axis=1)
    tc_out = _run_tc(dirs3, verts_b.T, verts6)
    out = lax.dynamic_update_slice(
        tc_out, sc_out.reshape(_B_SC, _OUT_W), (_B_TC, 0))
    return out.reshape(B, N_QUERY, 3)
